# baseline (speedup 1.0000x reference)
import jax
import jax.numpy as jnp
from jax.experimental import pallas as pl

_B, _H = 16384, 200


def _body(xx_ref, xy_ref, o_ref):
    scale = jnp.float32(2.0 / 3.0)
    x3 = xx_ref[...].reshape(8, 128, 128)
    y3 = xy_ref[...].reshape(8, 128, 128)
    fx = x3.astype(jnp.float32) * scale - 1.0
    fy = y3.astype(jnp.float32) * scale - 1.0
    o_ref[...] = jnp.stack((fx, fy), axis=2).reshape(2048, 128)


def kernel(x_x, x_y):
    xt = x_x.T
    yt = x_y.T
    ispec = pl.BlockSpec((8, _B), lambda i: (i, 0))
    ospec = pl.BlockSpec((2048, 128), lambda i: (i, 0))
    f = pl.pallas_call(
        _body,
        grid=(_H // 8,),
        in_specs=[ispec, ispec],
        out_specs=ospec,
        out_shape=jax.ShapeDtypeStruct((_H * 256, 128), jnp.float32),
    )(xt, yt)
    cf = f.astype(jnp.complex64)
    cf = jax.lax.optimization_barrier(cf)
    out = cf.reshape(_H, 128, 2, 128).transpose(1, 3, 0, 2)
    return out.reshape(_B, _H, 2)
